# baseline (speedup 1.0000x reference)
import functools

import jax
import jax.numpy as jnp
from jax import lax
from jax.experimental import pallas as pl
from jax.experimental.pallas import tpu as pltpu
from jax.experimental.pallas import tpu_sc as plsc

NC = 2
NS = 16
NW = NC * NS
L = 16
TCW = 128
CHB = 128


@functools.partial(jax.jit, static_argnums=(3,))
def _run_bias(pi2, ni2, beta1d, bpw):
    nch = pi2.shape[0] // NW
    mesh = plsc.VectorSubcoreMesh(
        core_axis_name="c", subcore_axis_name="s",
        num_cores=NC, num_subcores=NS)

    @functools.partial(
        pl.kernel,
        out_type=jax.ShapeDtypeStruct((NW * bpw,), jnp.float32),
        mesh=mesh,
        scratch_types=[
            pltpu.VMEM((nch, CHB), jnp.int32),
            pltpu.VMEM((nch, CHB), jnp.int32),
            pltpu.VMEM((bpw,), jnp.float32),
            pltpu.VMEM((bpw,), jnp.float32),
            pltpu.VMEM((bpw,), jnp.float32),
            pltpu.SemaphoreType.DMA,
        ],
        compiler_params=pltpu.CompilerParams(
            needs_layout_passes=False, use_tc_tiling_on_sc=False),
    )
    def k(pi_hbm, ni_hbm, bb_hbm, out_hbm, pi_v, ni_v, pb_v, nb_v, out_v,
          sem):
        wid = lax.axis_index("s") * NC + lax.axis_index("c")
        pltpu.sync_copy(pi_hbm.at[pl.ds(wid * nch, nch)], pi_v)
        pltpu.sync_copy(ni_hbm.at[pl.ds(wid * nch, nch)], ni_v)
        cps = []
        for j in range(nch):
            rows = pl.ds(j * CHB, CHB)
            cps.append(pltpu.async_copy(
                bb_hbm.at[pi_v.at[j]], pb_v.at[rows], sem))
            cps.append(pltpu.async_copy(
                bb_hbm.at[ni_v.at[j]], nb_v.at[rows], sem))
        for c in cps:
            c.wait()

        def group(g, carry):
            sl = pl.ds(g * L, L)
            out_v[sl] = pb_v[sl] - nb_v[sl]
            return carry

        lax.fori_loop(0, bpw // L, group, 0)
        pltpu.sync_copy(out_v, out_hbm.at[pl.ds(wid * bpw, bpw)])

    return k(pi2, ni2, beta1d)


@functools.partial(jax.jit, static_argnums=(8, 9, 10))
def _run_gather(gut, git, ru, su, ri, si, bu, bi, ntc, nu, ni_tot):
    dump_u = nu
    ibase = nu + L
    dump_i = ibase + ni_tot
    irows = dump_i + L
    tpw = (ntc + NW - 1) // NW
    bnd_n = ((tpw + 32) // 16) * 16
    mesh = plsc.VectorSubcoreMesh(
        core_axis_name="c", subcore_axis_name="s",
        num_cores=NC, num_subcores=NS)

    @functools.partial(
        pl.kernel,
        out_type=jax.ShapeDtypeStruct((irows, TCW), jnp.float32),
        mesh=mesh,
        scratch_types=[
            pltpu.VMEM((bnd_n,), jnp.int32),
            pltpu.VMEM((bnd_n,), jnp.int32),
            pltpu.VMEM((2, 64, TCW), jnp.float32),
            pltpu.VMEM((2, 64, TCW), jnp.float32),
            pltpu.VMEM((2, 64), jnp.int32),
            pltpu.VMEM((2, 64), jnp.int32),
            pltpu.VMEM((2, 64), jnp.int32),
            pltpu.VMEM((2, 64), jnp.int32),
            pltpu.VMEM((L, TCW), jnp.float32),
            pltpu.VMEM((1, L), jnp.int32),
            pltpu.SemaphoreType.DMA,
            pltpu.SemaphoreType.DMA,
            pltpu.SemaphoreType.DMA,
            pltpu.SemaphoreType.DMA,
            pltpu.SemaphoreType.DMA,
            pltpu.SemaphoreType.DMA,
        ],
        compiler_params=pltpu.CompilerParams(needs_layout_passes=False),
    )
    def k(gut_hbm, git_hbm, ru_hbm, su_hbm, ri_hbm, si_hbm, bu_hbm, bi_hbm,
          out_hbm, bndu, bndi, cbu, cbi, rbu, pbu, rbi, pbi,
          rowstage, pos2, semc0, semc1, sems0, sems1, semr, semw):
        wid = lax.axis_index("s") * NC + lax.axis_index("c")
        t0 = wid * tpw
        pltpu.sync_copy(bu_hbm.at[wid], bndu)
        pltpu.sync_copy(bi_hbm.at[wid], bndi)

        lanes = lax.iota(jnp.int32, L)
        semc = (semc0, semc1)
        sems = (sems0, sems1)

        def scal(v, lane):
            return jnp.sum(jnp.where(lanes == lane, v, 0))

        def vgather(ref, off):
            return plsc.load_gather(ref, [off + lanes])

        def fire_col(tc, par):
            off = pl.multiple_of(tc * TCW, TCW)
            pltpu.async_copy(gut_hbm.at[:, pl.ds(off, TCW)], cbu.at[par],
                             semc[par])
            pltpu.async_copy(git_hbm.at[:, pl.ds(off, TCW)], cbi.at[par],
                             semc[par])

        def fire_stage(lo_u, lo_i, par):
            au = pl.multiple_of((lo_u // 8) * 8, 8)
            ai = pl.multiple_of((lo_i // 8) * 8, 8)
            pltpu.async_copy(ru_hbm.at[pl.ds(au, 64)], rbu.at[par],
                             sems[par])
            pltpu.async_copy(su_hbm.at[pl.ds(au, 64)], pbu.at[par],
                             sems[par])
            pltpu.async_copy(ri_hbm.at[pl.ds(ai, 64)], rbi.at[par],
                             sems[par])
            pltpu.async_copy(si_hbm.at[pl.ds(ai, 64)], pbi.at[par],
                             sems[par])

        def drain(ref_src, dst, sem):
            pltpu.make_async_copy(ref_src, dst, sem).wait()

        def extract(cb, r_hbm, rv_ref, p_hbm, pv_ref, lo, hi, tc, par,
                    dump):
            cnt = hi - lo

            def rnd(r, carry):
                st = lo + r * 48
                skew = st - (st // 8) * 8

                @pl.when(r > 0)
                def _():
                    a = pl.multiple_of((st // 8) * 8, 8)
                    pltpu.async_copy(r_hbm.at[pl.ds(a, 64)],
                                     rv_ref.at[par], semr)
                    pltpu.async_copy(p_hbm.at[pl.ds(a, 64)],
                                     pv_ref.at[par], semr)
                    drain(r_hbm.at[pl.ds(0, 64)], rv_ref.at[par], semr)
                    drain(p_hbm.at[pl.ds(0, 64)], pv_ref.at[par], semr)

                for g3 in range(3):
                    goff = r * 48 + g3 * 16

                    @pl.when(goff < cnt)
                    def _():
                        rv = vgather(rv_ref.at[par], skew + g3 * L)
                        pv = vgather(pv_ref.at[par], skew + g3 * L)
                        rloc = rv - tc * TCW
                        vm = lanes < (cnt - goff)
                        for d in range(64):
                            fd = jnp.full((L,), d, jnp.int32)
                            v = plsc.load_gather(cb.at[par], [fd, rloc])
                            plsc.store_scatter(rowstage, [lanes, fd], v)
                        psel = jnp.where(vm, pv, dump + lanes)
                        pos2.at[0][...] = psel
                        pltpu.async_copy(
                            rowstage, out_hbm.at[pos2.at[0]], semw).wait()
                return carry

            lax.fori_loop(0, (cnt + 47) // 48, rnd, 0)

        bv_u0 = bndu[pl.ds(0, L)]
        bv_i0 = bndi[pl.ds(0, L)]
        fire_col(t0, 0)
        fire_stage(scal(bv_u0, 0), scal(bv_i0, 0), 0)

        def chunk2(c2, carry):
            for par in range(2):
                c = c2 * 2 + par
                tc = t0 + c
                bvu = vgather(bndu, c)
                bvi = vgather(bndi, c)
                lo_u = scal(bvu, 0)
                hi_u = scal(bvu, 1)
                lo_i = scal(bvi, 0)
                hi_i = scal(bvi, 1)

                @pl.when(jnp.logical_and(c + 1 < tpw, tc + 1 < ntc))
                def _():
                    fire_col(tc + 1, 1 - par)
                    fire_stage(hi_u, hi_i, 1 - par)

                @pl.when(jnp.logical_and(c < tpw, tc < ntc))
                def _():
                    drain(gut_hbm.at[:, pl.ds(0, TCW)], cbu.at[par],
                          semc[par])
                    drain(git_hbm.at[:, pl.ds(0, TCW)], cbi.at[par],
                          semc[par])
                    drain(ru_hbm.at[pl.ds(0, 64)], rbu.at[par], sems[par])
                    drain(su_hbm.at[pl.ds(0, 64)], pbu.at[par], sems[par])
                    drain(ri_hbm.at[pl.ds(0, 64)], rbi.at[par], sems[par])
                    drain(si_hbm.at[pl.ds(0, 64)], pbi.at[par], sems[par])
                    extract(cbu, ru_hbm, rbu, su_hbm, pbu,
                            lo_u, hi_u, tc, par, dump_u)
                    extract(cbi, ri_hbm, rbi, si_hbm, pbi,
                            lo_i, hi_i, tc, par, dump_i)
            return carry

        lax.fori_loop(0, (tpw + 1) // 2, chunk2, 0)

    return k(gut, git, ru, su, ri, si, bu, bi)


@functools.partial(jax.jit, static_argnums=(2, 3, 4))
def _run_dot(inter, bdiff, bpw, dim, ibase):
    npass = 2
    pb = bpw // npass
    nu = NW * bpw
    mesh = plsc.VectorSubcoreMesh(
        core_axis_name="c", subcore_axis_name="s",
        num_cores=NC, num_subcores=NS)

    @functools.partial(
        pl.kernel,
        out_type=jax.ShapeDtypeStruct((NW * bpw,), jnp.float32),
        mesh=mesh,
        scratch_types=[
            pltpu.VMEM((pb, TCW), jnp.float32),
            pltpu.VMEM((pb, TCW), jnp.float32),
            pltpu.VMEM((pb, TCW), jnp.float32),
            pltpu.VMEM((bpw,), jnp.float32),
            pltpu.VMEM((bpw,), jnp.float32),
            pltpu.SemaphoreType.DMA,
        ],
        compiler_params=pltpu.CompilerParams(needs_layout_passes=False),
    )
    def k(it_hbm, bd_hbm, out_hbm, urows, prows, nrows, bd_v, out_v, sem):
        wid = lax.axis_index("s") * NC + lax.axis_index("c")
        base = wid * bpw
        pltpu.sync_copy(bd_hbm.at[pl.ds(base, bpw)], bd_v)

        lanes = lax.iota(jnp.int32, L)
        zf = jnp.zeros((L,), jnp.float32)

        for p in range(npass):
            b0 = base + p * pb
            cps = [
                pltpu.async_copy(
                    it_hbm.at[pl.ds(b0, pb)], urows, sem),
                pltpu.async_copy(
                    it_hbm.at[pl.ds(ibase + b0, pb)], prows, sem),
                pltpu.async_copy(
                    it_hbm.at[pl.ds(ibase + nu + b0, pb)], nrows, sem),
            ]
            for c in cps:
                c.wait()

            def group(g, carry):
                glb = pl.ds(p * pb + g * L, L)
                lidx = g * L + lanes
                accs = [zf, zf, zf, zf]
                for d in range(dim):
                    fd = jnp.full((L,), d, jnp.int32)
                    u = plsc.load_gather(urows, [lidx, fd])
                    pr = plsc.load_gather(prows, [lidx, fd])
                    n = plsc.load_gather(nrows, [lidx, fd])
                    accs[d % 4] = accs[d % 4] + u * (pr - n)
                out_v[glb] = (accs[0] + accs[1]) + (accs[2] + accs[3]) \
                    + bd_v[glb]
                return carry

            lax.fori_loop(0, pb // L, group, 0)

        pltpu.sync_copy(out_v, out_hbm.at[pl.ds(base, bpw)])

    return k(inter, bdiff)


def kernel(ui, pi, ni, gamma_users, gamma_items, beta_items):
    b = ui.shape[0]
    rows, dim = gamma_users.shape
    bpw = b // NW
    nch = bpw // CHB
    ntc = (rows + TCW - 1) // TCW
    ui32 = ui.astype(jnp.int32)
    pi32 = pi.astype(jnp.int32)
    ni32 = ni.astype(jnp.int32)
    gut = gamma_users.T
    git = gamma_items.T
    beta1d = beta_items.reshape(-1)

    su = jnp.argsort(ui32).astype(jnp.int32)
    ru = ui32[su]
    ci = jnp.concatenate([pi32, ni32])
    si = jnp.argsort(ci).astype(jnp.int32)
    ri = ci[si]
    edges = (jnp.arange(ntc + 1, dtype=jnp.int32) * TCW)
    bu = jnp.searchsorted(ru, edges).astype(jnp.int32)
    bi = jnp.searchsorted(ri, edges).astype(jnp.int32)
    pad = 128
    dump_u = b
    ibase = b + L
    dump_i = ibase + 2 * b
    ru_p = jnp.concatenate([ru, jnp.zeros((pad,), jnp.int32)])
    su_p = jnp.concatenate([su, jnp.full((pad,), dump_u, jnp.int32)])
    ri_p = jnp.concatenate([ri, jnp.zeros((pad,), jnp.int32)])
    si_p = jnp.concatenate(
        [si + ibase, jnp.full((pad,), dump_i, jnp.int32)])
    tpw = (ntc + NW - 1) // NW
    bnd_n = ((tpw + 32) // 16) * 16
    bu_l = jnp.concatenate([bu, jnp.full((bnd_n,), b, jnp.int32)])
    bi_l = jnp.concatenate([bi, jnp.full((bnd_n,), 2 * b, jnp.int32)])
    widx = (jnp.arange(NW, dtype=jnp.int32)[:, None] * tpw
            + jnp.arange(bnd_n, dtype=jnp.int32)[None, :])
    bu_p = bu_l[widx]
    bi_p = bi_l[widx]

    pi2 = pi32.reshape(NW * nch, CHB)
    ni2 = ni32.reshape(NW * nch, CHB)
    bdiff = _run_bias(pi2, ni2, beta1d, bpw)
    inter = _run_gather(gut, git, ru_p, su_p, ri_p, si_p, bu_p, bi_p,
                        ntc, b, 2 * b)
    out = _run_dot(inter, bdiff, bpw, dim, ibase)
    return out.reshape(b, 1, 1)


# speedup vs baseline: 1.7514x; 1.7514x over previous
import functools

import jax
import jax.numpy as jnp
from jax import lax
from jax.experimental import pallas as pl
from jax.experimental.pallas import tpu as pltpu
from jax.experimental.pallas import tpu_sc as plsc

NC = 2
NS = 16
NW = NC * NS
L = 16
TCW = 128
CHB = 128


@functools.partial(jax.jit, static_argnums=(3,))
def _run_bias(pi2, ni2, beta1d, bpw):
    nch = pi2.shape[0] // NW
    mesh = plsc.VectorSubcoreMesh(
        core_axis_name="c", subcore_axis_name="s",
        num_cores=NC, num_subcores=NS)

    @functools.partial(
        pl.kernel,
        out_type=jax.ShapeDtypeStruct((NW * bpw,), jnp.float32),
        mesh=mesh,
        scratch_types=[
            pltpu.VMEM((nch, CHB), jnp.int32),
            pltpu.VMEM((nch, CHB), jnp.int32),
            pltpu.VMEM((bpw,), jnp.float32),
            pltpu.VMEM((bpw,), jnp.float32),
            pltpu.VMEM((bpw,), jnp.float32),
            pltpu.SemaphoreType.DMA,
        ],
        compiler_params=pltpu.CompilerParams(
            needs_layout_passes=False, use_tc_tiling_on_sc=False),
    )
    def k(pi_hbm, ni_hbm, bb_hbm, out_hbm, pi_v, ni_v, pb_v, nb_v, out_v,
          sem):
        wid = lax.axis_index("s") * NC + lax.axis_index("c")
        pltpu.sync_copy(pi_hbm.at[pl.ds(wid * nch, nch)], pi_v)
        pltpu.sync_copy(ni_hbm.at[pl.ds(wid * nch, nch)], ni_v)
        cps = []
        for j in range(nch):
            rows = pl.ds(j * CHB, CHB)
            cps.append(pltpu.async_copy(
                bb_hbm.at[pi_v.at[j]], pb_v.at[rows], sem))
            cps.append(pltpu.async_copy(
                bb_hbm.at[ni_v.at[j]], nb_v.at[rows], sem))
        for c in cps:
            c.wait()

        def group(g, carry):
            sl = pl.ds(g * L, L)
            out_v[sl] = pb_v[sl] - nb_v[sl]
            return carry

        lax.fori_loop(0, bpw // L, group, 0)
        pltpu.sync_copy(out_v, out_hbm.at[pl.ds(wid * bpw, bpw)])

    return k(pi2, ni2, beta1d)


TCB = 256


@functools.partial(jax.jit, static_argnums=(8, 9, 10))
def _run_gather(gut, git, ru, su, ri, si, bu, bi, ntc, nu, ni_tot):
    dump_u = nu
    ibase = nu + 48
    dump_i = ibase + ni_tot
    irows = dump_i + 48
    tpw = (ntc + NW - 1) // NW
    bnd_n = ((tpw + 32) // 16) * 16
    rows_pad = ((gut.shape[1] + 127) // 128) * 128
    mesh = plsc.VectorSubcoreMesh(
        core_axis_name="c", subcore_axis_name="s",
        num_cores=NC, num_subcores=NS)

    @functools.partial(
        pl.kernel,
        out_type=jax.ShapeDtypeStruct((irows, TCW), jnp.float32),
        mesh=mesh,
        scratch_types=[
            pltpu.VMEM((bnd_n,), jnp.int32),
            pltpu.VMEM((bnd_n,), jnp.int32),
            pltpu.VMEM((2, 64, TCB), jnp.float32),
            pltpu.VMEM((2, 64, TCB), jnp.float32),
            pltpu.VMEM((2, 64), jnp.int32),
            pltpu.VMEM((2, 64), jnp.int32),
            pltpu.VMEM((2, 64), jnp.int32),
            pltpu.VMEM((2, 64), jnp.int32),
            pltpu.VMEM((2, 48, TCW), jnp.float32),
            pltpu.VMEM((2, 48, TCW), jnp.float32),
            pltpu.VMEM((2, 48), jnp.int32),
            pltpu.VMEM((2, 48), jnp.int32),
            pltpu.SemaphoreType.DMA,
            pltpu.SemaphoreType.DMA,
            pltpu.SemaphoreType.DMA,
            pltpu.SemaphoreType.DMA,
            pltpu.SemaphoreType.DMA,
            pltpu.SemaphoreType.DMA,
            pltpu.SemaphoreType.DMA,
            pltpu.SemaphoreType.DMA,
            pltpu.SemaphoreType.DMA,
        ],
        compiler_params=pltpu.CompilerParams(needs_layout_passes=False),
    )
    def k(gut_hbm, git_hbm, ru_hbm, su_hbm, ri_hbm, si_hbm, bu_hbm, bi_hbm,
          out_hbm, bndu, bndi, cbu, cbi, rbu, pbu, rbi, pbi,
          rsu, rsi, posu, posi,
          semc0, semc1, sems0, sems1, semr, semw00, semw01, semw10, semw11):
        wid = lax.axis_index("s") * NC + lax.axis_index("c")
        t0 = wid * tpw
        pltpu.sync_copy(bu_hbm.at[wid], bndu)
        pltpu.sync_copy(bi_hbm.at[wid], bndi)

        lanes = lax.iota(jnp.int32, L)
        semc = (semc0, semc1)
        sems = (sems0, sems1)

        def scal(v, lane):
            return jnp.sum(jnp.where(lanes == lane, v, 0))

        def vgather(ref, off):
            return plsc.load_gather(ref, [off + lanes])

        semw = ((semw00, semw01), (semw10, semw11))

        def coffset(tc):
            return pl.multiple_of(
                jnp.minimum(tc * TCB, rows_pad - TCB), TCW)

        def fire_col(tc, par):
            off = coffset(tc)
            pltpu.async_copy(gut_hbm.at[:, pl.ds(off, TCB)], cbu.at[par],
                             semc[par])
            pltpu.async_copy(git_hbm.at[:, pl.ds(off, TCB)], cbi.at[par],
                             semc[par])

        def fire_stage(lo_u, lo_i, par):
            au = pl.multiple_of((lo_u // 8) * 8, 8)
            ai = pl.multiple_of((lo_i // 8) * 8, 8)
            pltpu.async_copy(ru_hbm.at[pl.ds(au, 64)], rbu.at[par],
                             sems[par])
            pltpu.async_copy(su_hbm.at[pl.ds(au, 64)], pbu.at[par],
                             sems[par])
            pltpu.async_copy(ri_hbm.at[pl.ds(ai, 64)], rbi.at[par],
                             sems[par])
            pltpu.async_copy(si_hbm.at[pl.ds(ai, 64)], pbi.at[par],
                             sems[par])

        def drain(ref_src, dst, sem):
            pltpu.make_async_copy(ref_src, dst, sem).wait()

        def extract(cb, r_hbm, rv_ref, p_hbm, pv_ref, lo, hi, coff, par,
                    dump, rs, pos, semw_s, c2):
            cnt = hi - lo

            @pl.when(c2 >= 1)
            def _():
                drain(rs.at[par], out_hbm.at[pos.at[par]], semw_s[par])

            def rnd(r, carry):
                st = lo + r * 48
                skew = st - (st // 8) * 8

                @pl.when(r > 0)
                def _():
                    a = pl.multiple_of((st // 8) * 8, 8)
                    pltpu.async_copy(r_hbm.at[pl.ds(a, 64)],
                                     rv_ref.at[par], semr)
                    pltpu.async_copy(p_hbm.at[pl.ds(a, 64)],
                                     pv_ref.at[par], semr)
                    drain(r_hbm.at[pl.ds(0, 64)], rv_ref.at[par], semr)
                    drain(p_hbm.at[pl.ds(0, 64)], pv_ref.at[par], semr)
                    drain(rs.at[par], out_hbm.at[pos.at[par]], semw_s[par])

                for g3 in range(3):
                    goff = r * 48 + g3 * 16
                    rv = vgather(rv_ref.at[par], skew + g3 * L)
                    pv = vgather(pv_ref.at[par], skew + g3 * L)
                    rloc = rv - coff
                    vm = (goff + lanes) < cnt
                    rloc = jnp.where(vm, rloc, 0)

                    @pl.when(goff < cnt)
                    def _():
                        for d in range(64):
                            fd = jnp.full((L,), d, jnp.int32)
                            v = plsc.load_gather(cb.at[par], [fd, rloc])
                            plsc.store_scatter(
                                rs.at[par], [g3 * L + lanes, fd], v)
                    psel = jnp.where(vm, pv, dump + g3 * L + lanes)
                    pos.at[par][pl.ds(g3 * L, L)] = psel
                pltpu.async_copy(rs.at[par], out_hbm.at[pos.at[par]],
                                 semw_s[par])
                return carry

            lax.fori_loop(0, jnp.maximum((cnt + 47) // 48, 1), rnd, 0)

        bv_u0 = bndu[pl.ds(0, L)]
        bv_i0 = bndi[pl.ds(0, L)]
        fire_col(t0, 0)
        fire_stage(scal(bv_u0, 0), scal(bv_i0, 0), 0)

        def chunk2(c2, carry):
            for par in range(2):
                c = c2 * 2 + par
                tc = t0 + c

                @pl.when(c < tpw)
                def _():
                    bvu = vgather(bndu, c)
                    bvi = vgather(bndi, c)
                    lo_u = scal(bvu, 0)
                    hi_u = scal(bvu, 1)
                    lo_i = scal(bvi, 0)
                    hi_i = scal(bvi, 1)

                    @pl.when(c + 1 < tpw)
                    def _():
                        fire_col(tc + 1, 1 - par)
                        fire_stage(hi_u, hi_i, 1 - par)

                    drain(gut_hbm.at[:, pl.ds(0, TCB)], cbu.at[par],
                          semc[par])
                    drain(git_hbm.at[:, pl.ds(0, TCB)], cbi.at[par],
                          semc[par])
                    drain(ru_hbm.at[pl.ds(0, 64)], rbu.at[par], sems[par])
                    drain(su_hbm.at[pl.ds(0, 64)], pbu.at[par], sems[par])
                    drain(ri_hbm.at[pl.ds(0, 64)], rbi.at[par], sems[par])
                    drain(si_hbm.at[pl.ds(0, 64)], pbi.at[par], sems[par])
                    coff = coffset(tc)
                    extract(cbu, ru_hbm, rbu, su_hbm, pbu,
                            lo_u, hi_u, coff, par, dump_u,
                            rsu, posu, semw[0], c2)
                    extract(cbi, ri_hbm, rbi, si_hbm, pbi,
                            lo_i, hi_i, coff, par, dump_i,
                            rsi, posi, semw[1], c2)
            return carry

        lax.fori_loop(0, (tpw + 1) // 2, chunk2, 0)
        for par in range(2):
            drain(rsu.at[par], out_hbm.at[posu.at[par]], semw[0][par])
            drain(rsi.at[par], out_hbm.at[posi.at[par]], semw[1][par])

    return k(gut, git, ru, su, ri, si, bu, bi)


@functools.partial(jax.jit, static_argnums=(2, 3, 4))
def _run_dot(inter, bdiff, bpw, dim, ibase):
    npass = 2
    pb = bpw // npass
    nu = NW * bpw
    mesh = plsc.VectorSubcoreMesh(
        core_axis_name="c", subcore_axis_name="s",
        num_cores=NC, num_subcores=NS)

    @functools.partial(
        pl.kernel,
        out_type=jax.ShapeDtypeStruct((NW * bpw,), jnp.float32),
        mesh=mesh,
        scratch_types=[
            pltpu.VMEM((pb, TCW), jnp.float32),
            pltpu.VMEM((pb, TCW), jnp.float32),
            pltpu.VMEM((pb, TCW), jnp.float32),
            pltpu.VMEM((bpw,), jnp.float32),
            pltpu.VMEM((bpw,), jnp.float32),
            pltpu.SemaphoreType.DMA,
        ],
        compiler_params=pltpu.CompilerParams(needs_layout_passes=False),
    )
    def k(it_hbm, bd_hbm, out_hbm, urows, prows, nrows, bd_v, out_v, sem):
        wid = lax.axis_index("s") * NC + lax.axis_index("c")
        base = wid * bpw
        pltpu.sync_copy(bd_hbm.at[pl.ds(base, bpw)], bd_v)

        lanes = lax.iota(jnp.int32, L)
        zf = jnp.zeros((L,), jnp.float32)

        for p in range(npass):
            b0 = base + p * pb
            cps = [
                pltpu.async_copy(
                    it_hbm.at[pl.ds(b0, pb)], urows, sem),
                pltpu.async_copy(
                    it_hbm.at[pl.ds(ibase + b0, pb)], prows, sem),
                pltpu.async_copy(
                    it_hbm.at[pl.ds(ibase + nu + b0, pb)], nrows, sem),
            ]
            for c in cps:
                c.wait()

            def group(g, carry):
                glb = pl.ds(p * pb + g * L, L)
                lidx = g * L + lanes
                accs = [zf, zf, zf, zf]
                for d in range(dim):
                    fd = jnp.full((L,), d, jnp.int32)
                    u = plsc.load_gather(urows, [lidx, fd])
                    pr = plsc.load_gather(prows, [lidx, fd])
                    n = plsc.load_gather(nrows, [lidx, fd])
                    accs[d % 4] = accs[d % 4] + u * (pr - n)
                out_v[glb] = (accs[0] + accs[1]) + (accs[2] + accs[3]) \
                    + bd_v[glb]
                return carry

            lax.fori_loop(0, pb // L, group, 0)

        pltpu.sync_copy(out_v, out_hbm.at[pl.ds(base, bpw)])

    return k(inter, bdiff)


def kernel(ui, pi, ni, gamma_users, gamma_items, beta_items):
    b = ui.shape[0]
    rows, dim = gamma_users.shape
    bpw = b // NW
    nch = bpw // CHB
    ntc = (rows + TCB - 1) // TCB
    ui32 = ui.astype(jnp.int32)
    pi32 = pi.astype(jnp.int32)
    ni32 = ni.astype(jnp.int32)
    gut = gamma_users.T
    git = gamma_items.T
    beta1d = beta_items.reshape(-1)

    su = jnp.argsort(ui32).astype(jnp.int32)
    ru = ui32[su]
    ci = jnp.concatenate([pi32, ni32])
    si = jnp.argsort(ci).astype(jnp.int32)
    ri = ci[si]
    edges = (jnp.arange(ntc + 1, dtype=jnp.int32) * TCB)
    bu = jnp.searchsorted(ru, edges).astype(jnp.int32)
    bi = jnp.searchsorted(ri, edges).astype(jnp.int32)
    pad = 128
    dump_u = b
    ibase = b + 48
    dump_i = ibase + 2 * b
    ru_p = jnp.concatenate([ru, jnp.zeros((pad,), jnp.int32)])
    su_p = jnp.concatenate([su, jnp.full((pad,), dump_u, jnp.int32)])
    ri_p = jnp.concatenate([ri, jnp.zeros((pad,), jnp.int32)])
    si_p = jnp.concatenate(
        [si + ibase, jnp.full((pad,), dump_i, jnp.int32)])
    tpw = (ntc + NW - 1) // NW
    bnd_n = ((tpw + 32) // 16) * 16
    bu_l = jnp.concatenate([bu, jnp.full((bnd_n,), b, jnp.int32)])
    bi_l = jnp.concatenate([bi, jnp.full((bnd_n,), 2 * b, jnp.int32)])
    widx = (jnp.arange(NW, dtype=jnp.int32)[:, None] * tpw
            + jnp.arange(bnd_n, dtype=jnp.int32)[None, :])
    bu_p = bu_l[widx]
    bi_p = bi_l[widx]

    pi2 = pi32.reshape(NW * nch, CHB)
    ni2 = ni32.reshape(NW * nch, CHB)
    bdiff = _run_bias(pi2, ni2, beta1d, bpw)
    inter = _run_gather(gut, git, ru_p, su_p, ri_p, si_p, bu_p, bi_p,
                        ntc, b, 2 * b)
    out = _run_dot(inter, bdiff, bpw, dim, ibase)
    return out.reshape(b, 1, 1)


# speedup vs baseline: 2.0804x; 1.1878x over previous
import functools

import jax
import jax.numpy as jnp
from jax import lax
from jax.experimental import pallas as pl
from jax.experimental.pallas import tpu as pltpu
from jax.experimental.pallas import tpu_sc as plsc

NC = 2
NS = 16
NW = NC * NS
L = 16


@functools.partial(jax.jit, static_argnums=(6, 7))
def _run(ui2, pi2, ni2, gamma_users, gamma_items, beta_items, dim, bpw):
    nch = ui2.shape[0] // NW
    chb = ui2.shape[1]
    ng = bpw // L
    mesh = plsc.VectorSubcoreMesh(
        core_axis_name="c", subcore_axis_name="s",
        num_cores=NC, num_subcores=NS)

    @functools.partial(
        pl.kernel,
        out_type=jax.ShapeDtypeStruct((NW * bpw,), jnp.float32),
        mesh=mesh,
        scratch_types=[
            pltpu.VMEM((nch, chb), jnp.int32),
            pltpu.VMEM((nch, chb), jnp.int32),
            pltpu.VMEM((nch, chb), jnp.int32),
            pltpu.VMEM((bpw, dim), jnp.float32),
            pltpu.VMEM((bpw, dim), jnp.float32),
            pltpu.VMEM((bpw, dim), jnp.float32),
            pltpu.VMEM((bpw,), jnp.float32),
            pltpu.VMEM((bpw,), jnp.float32),
            pltpu.VMEM((bpw,), jnp.float32),
            pltpu.SemaphoreType.DMA,
        ],
        compiler_params=pltpu.CompilerParams(
            needs_layout_passes=False, use_tc_tiling_on_sc=False),
    )
    def k(ui_hbm, pi_hbm, ni_hbm, gu_hbm, gi_hbm, bb_hbm, out_hbm,
          ui_v, pi_v, ni_v, urows, prows, nrows, pb_v, nb_v, out_v, sem):
        wid = lax.axis_index("s") * NC + lax.axis_index("c")
        base = wid * bpw

        pltpu.sync_copy(ui_hbm.at[pl.ds(wid * nch, nch)], ui_v)
        pltpu.sync_copy(pi_hbm.at[pl.ds(wid * nch, nch)], pi_v)
        pltpu.sync_copy(ni_hbm.at[pl.ds(wid * nch, nch)], ni_v)

        cps = []
        for j in range(nch):
            rows = pl.ds(j * chb, chb)
            cps.append(pltpu.async_copy(gu_hbm.at[ui_v.at[j]], urows.at[rows], sem))
            cps.append(pltpu.async_copy(gi_hbm.at[pi_v.at[j]], prows.at[rows], sem))
            cps.append(pltpu.async_copy(gi_hbm.at[ni_v.at[j]], nrows.at[rows], sem))
            cps.append(pltpu.async_copy(bb_hbm.at[pi_v.at[j]], pb_v.at[rows], sem))
            cps.append(pltpu.async_copy(bb_hbm.at[ni_v.at[j]], nb_v.at[rows], sem))
        for c in cps:
            c.wait()

        zf = jnp.zeros((L,), jnp.float32)

        def group(g, carry):
            idx0 = g * L + lax.iota(jnp.int32, L)
            pb = pb_v[pl.ds(g * L, L)]
            nb = nb_v[pl.ds(g * L, L)]
            accs = [zf, zf, zf, zf]
            for d in range(dim):
                idxd = jnp.full((L,), d, jnp.int32)
                u = plsc.load_gather(urows, [idx0, idxd])
                p = plsc.load_gather(prows, [idx0, idxd])
                n = plsc.load_gather(nrows, [idx0, idxd])
                accs[d % 4] = accs[d % 4] + u * (p - n)
            res = (accs[0] + accs[1]) + (accs[2] + accs[3]) + pb - nb
            out_v[pl.ds(g * L, L)] = res
            return carry

        lax.fori_loop(0, ng, group, 0)
        pltpu.sync_copy(out_v, out_hbm.at[pl.ds(base, bpw)])

    return k(ui2, pi2, ni2, gamma_users, gamma_items, beta_items)


def kernel(ui, pi, ni, gamma_users, gamma_items, beta_items):
    b = ui.shape[0]
    dim = gamma_users.shape[1]
    bpw = b // NW
    chb = min(128, bpw)
    nch = bpw // chb
    ui2 = ui.astype(jnp.int32).reshape(NW * nch, chb)
    pi2 = pi.astype(jnp.int32).reshape(NW * nch, chb)
    ni2 = ni.astype(jnp.int32).reshape(NW * nch, chb)
    beta1d = beta_items.reshape(-1)
    out = _run(ui2, pi2, ni2, gamma_users, gamma_items, beta1d, dim, bpw)
    return out.reshape(b, 1, 1)


# speedup vs baseline: 2.1193x; 1.0187x over previous
import functools

import jax
import jax.numpy as jnp
from jax import lax
from jax.experimental import pallas as pl
from jax.experimental.pallas import tpu as pltpu
from jax.experimental.pallas import tpu_sc as plsc

NC = 2
NS = 16
NW = NC * NS
L = 16


@functools.partial(jax.jit, static_argnums=(4, 5))
def _run_items(pi2, ni2, gamma_items, beta1d, dim, bpw):
    nch = pi2.shape[0] // NW
    chb = pi2.shape[1]
    ng = bpw // L
    mesh = plsc.VectorSubcoreMesh(
        core_axis_name="c", subcore_axis_name="s",
        num_cores=NC, num_subcores=NS)

    @functools.partial(
        pl.kernel,
        out_type=(
            jax.ShapeDtypeStruct((NW * bpw, dim), jnp.float32),
            jax.ShapeDtypeStruct((NW * bpw,), jnp.float32),
        ),
        mesh=mesh,
        scratch_types=[
            pltpu.VMEM((nch, chb), jnp.int32),
            pltpu.VMEM((nch, chb), jnp.int32),
            pltpu.VMEM((bpw, dim), jnp.float32),
            pltpu.VMEM((bpw, dim), jnp.float32),
            pltpu.VMEM((bpw, dim), jnp.float32),
            pltpu.VMEM((bpw,), jnp.float32),
            pltpu.VMEM((bpw,), jnp.float32),
            pltpu.VMEM((bpw,), jnp.float32),
            pltpu.SemaphoreType.DMA,
        ],
        compiler_params=pltpu.CompilerParams(
            needs_layout_passes=False, use_tc_tiling_on_sc=False),
    )
    def k(pi_hbm, ni_hbm, gi_hbm, bb_hbm, dd_hbm, bd_hbm,
          pi_v, ni_v, prows, nrows, drows, pb_v, nb_v, bd_v, sem):
        wid = lax.axis_index("s") * NC + lax.axis_index("c")
        base = wid * bpw

        pltpu.sync_copy(pi_hbm.at[pl.ds(wid * nch, nch)], pi_v)
        pltpu.sync_copy(ni_hbm.at[pl.ds(wid * nch, nch)], ni_v)

        cps = []
        for j in range(nch):
            rows = pl.ds(j * chb, chb)
            cps.append(pltpu.async_copy(
                gi_hbm.at[pi_v.at[j]], prows.at[rows], sem))
            cps.append(pltpu.async_copy(
                gi_hbm.at[ni_v.at[j]], nrows.at[rows], sem))
            cps.append(pltpu.async_copy(
                bb_hbm.at[pi_v.at[j]], pb_v.at[rows], sem))
            cps.append(pltpu.async_copy(
                bb_hbm.at[ni_v.at[j]], nb_v.at[rows], sem))
        for c in cps:
            c.wait()

        lanes = lax.iota(jnp.int32, L)

        def group(g, carry):
            idx0 = g * L + lanes
            sl = pl.ds(g * L, L)
            bd_v[sl] = pb_v[sl] - nb_v[sl]
            for d in range(dim):
                idxd = jnp.full((L,), d, jnp.int32)
                p = plsc.load_gather(prows, [idx0, idxd])
                n = plsc.load_gather(nrows, [idx0, idxd])
                plsc.store_scatter(drows, [idx0, idxd], p - n)
            return carry

        lax.fori_loop(0, ng, group, 0)
        pltpu.sync_copy(drows, dd_hbm.at[pl.ds(base, bpw)])
        pltpu.sync_copy(bd_v, bd_hbm.at[pl.ds(base, bpw)])

    return k(pi2, ni2, gamma_items, beta1d)


@functools.partial(jax.jit, static_argnums=(4, 5))
def _run_users(ui2, gamma_users, dd, bd, dim, bpw):
    nch = ui2.shape[0] // NW
    chb = ui2.shape[1]
    ng = bpw // L
    mesh = plsc.VectorSubcoreMesh(
        core_axis_name="c", subcore_axis_name="s",
        num_cores=NC, num_subcores=NS)

    @functools.partial(
        pl.kernel,
        out_type=jax.ShapeDtypeStruct((NW * bpw,), jnp.float32),
        mesh=mesh,
        scratch_types=[
            pltpu.VMEM((nch, chb), jnp.int32),
            pltpu.VMEM((bpw, dim), jnp.float32),
            pltpu.VMEM((bpw, dim), jnp.float32),
            pltpu.VMEM((bpw,), jnp.float32),
            pltpu.VMEM((bpw,), jnp.float32),
            pltpu.SemaphoreType.DMA,
        ],
        compiler_params=pltpu.CompilerParams(
            needs_layout_passes=False, use_tc_tiling_on_sc=False),
    )
    def k(ui_hbm, gu_hbm, dd_hbm, bd_hbm, out_hbm,
          ui_v, urows, ddv, bd_v, out_v, sem):
        wid = lax.axis_index("s") * NC + lax.axis_index("c")
        base = wid * bpw

        pltpu.sync_copy(ui_hbm.at[pl.ds(wid * nch, nch)], ui_v)
        pltpu.sync_copy(dd_hbm.at[pl.ds(base, bpw)], ddv)
        pltpu.sync_copy(bd_hbm.at[pl.ds(base, bpw)], bd_v)

        cps = []
        for j in range(nch):
            rows = pl.ds(j * chb, chb)
            cps.append(pltpu.async_copy(
                gu_hbm.at[ui_v.at[j]], urows.at[rows], sem))
        for c in cps:
            c.wait()

        lanes = lax.iota(jnp.int32, L)
        zf = jnp.zeros((L,), jnp.float32)

        def group(g, carry):
            idx0 = g * L + lanes
            sl = pl.ds(g * L, L)
            accs = [zf, zf, zf, zf]
            for d in range(dim):
                idxd = jnp.full((L,), d, jnp.int32)
                u = plsc.load_gather(urows, [idx0, idxd])
                dv = plsc.load_gather(ddv, [idx0, idxd])
                accs[d % 4] = accs[d % 4] + u * dv
            out_v[sl] = (accs[0] + accs[1]) + (accs[2] + accs[3]) \
                + bd_v[sl]
            return carry

        lax.fori_loop(0, ng, group, 0)
        pltpu.sync_copy(out_v, out_hbm.at[pl.ds(base, bpw)])

    return k(ui2, gamma_users, dd, bd)


def kernel(ui, pi, ni, gamma_users, gamma_items, beta_items):
    b = ui.shape[0]
    dim = gamma_users.shape[1]
    bpw = b // NW
    chb = min(128, bpw)
    nch = bpw // chb
    ui2 = ui.astype(jnp.int32).reshape(NW * nch, chb)
    pi2 = pi.astype(jnp.int32).reshape(NW * nch, chb)
    ni2 = ni.astype(jnp.int32).reshape(NW * nch, chb)
    beta1d = beta_items.reshape(-1)
    dd, bd = _run_items(pi2, ni2, gamma_items, beta1d, dim, bpw)
    out = _run_users(ui2, gamma_users, dd, bd, dim, bpw)
    return out.reshape(b, 1, 1)
